# baseline (speedup 1.0000x reference)
import functools

import jax
import jax.numpy as jnp
from jax import lax
from jax.experimental import pallas as pl
from jax.experimental.pallas import tpu as pltpu
from jax.experimental.pallas import tpu_sc as plsc


def _make_sc_kernel(R, T, D):
    info = plsc.get_sparse_core_info()
    NC, NS, L = info.num_cores, info.num_subcores, info.num_lanes
    NW = NC * NS
    rows_per_w = R // NW
    CR = 32
    n_chunks = rows_per_w // CR
    n_col = D // L

    mesh = plsc.VectorSubcoreMesh(core_axis_name="c", subcore_axis_name="s")

    @functools.partial(
        pl.kernel,
        mesh=mesh,
        out_type=jax.ShapeDtypeStruct((R, D), jnp.float32),
        scratch_types=[
            pltpu.VMEM((CR, D), jnp.float32),
            pltpu.VMEM((CR, D), jnp.float32),
        ],
    )
    def k(x_hbm, pe_hbm, out_hbm, bufx, bufp):
        wid = lax.axis_index("s") * NC + lax.axis_index("c")
        base = wid * rows_per_w

        def chunk_body(i, carry):
            r0 = base + i * CR
            t0 = lax.rem(r0, T)
            pltpu.sync_copy(x_hbm.at[pl.ds(r0, CR)], bufx)
            pltpu.sync_copy(pe_hbm.at[pl.ds(t0, CR)], bufp)

            def row_body(r, carry2):
                for c in range(n_col):
                    v = bufp[r, pl.ds(c * L, L)]
                    plsc.addupdate(bufx.at[r, pl.ds(c * L, L)], v)
                return carry2

            lax.fori_loop(0, CR, row_body, 0)
            pltpu.sync_copy(bufx, out_hbm.at[pl.ds(r0, CR)])
            return carry

        lax.fori_loop(0, n_chunks, chunk_body, 0)

    return k


def kernel(x, pe_weight):
    B, T, D = x.shape
    R = B * T
    k = _make_sc_kernel(R, T, D)
    out = k(x.reshape(R, D), pe_weight)
    return out.reshape(B, T, D)


# speedup vs baseline: 1.6151x; 1.6151x over previous
import functools

import jax
import jax.numpy as jnp
from jax import lax
from jax.experimental import pallas as pl
from jax.experimental.pallas import tpu as pltpu
from jax.experimental.pallas import tpu_sc as plsc

_NB = 4
_LA = 2


def _make_sc_kernel(R, T, D):
    info = plsc.get_sparse_core_info()
    NC, NS, L = info.num_cores, info.num_subcores, info.num_lanes
    NW = NC * NS
    rows_per_w = R // NW
    CR = 16
    n_chunks = rows_per_w // CR
    n_col = D // L

    mesh = plsc.VectorSubcoreMesh(core_axis_name="c", subcore_axis_name="s")

    scratch = (
        [pltpu.VMEM((CR, D), jnp.float32) for _ in range(_NB)]
        + [pltpu.VMEM((CR, D), jnp.float32) for _ in range(_NB)]
        + [pltpu.SemaphoreType.DMA for _ in range(3 * _NB)]
    )

    @functools.partial(
        pl.kernel,
        mesh=mesh,
        out_type=jax.ShapeDtypeStruct((R, D), jnp.float32),
        scratch_types=scratch,
    )
    def k(x_hbm, pe_hbm, out_hbm, *refs):
        bufx = refs[:_NB]
        bufp = refs[_NB:2 * _NB]
        sem_lx = refs[2 * _NB:2 * _NB + _NB]
        sem_lp = refs[2 * _NB + _NB:2 * _NB + 2 * _NB]
        sem_st = refs[2 * _NB + 2 * _NB:]

        wid = lax.axis_index("s") * NC + lax.axis_index("c")
        base = wid * rows_per_w

        def start_loads(i, b):
            r0 = base + i * CR
            t0 = lax.rem(r0, T)
            pltpu.async_copy(x_hbm.at[pl.ds(r0, CR)], bufx[b], sem_lx[b])
            pltpu.async_copy(pe_hbm.at[pl.ds(t0, CR)], bufp[b], sem_lp[b])

        def wait_loads(i, b):
            r0 = base + i * CR
            t0 = lax.rem(r0, T)
            pltpu.make_async_copy(x_hbm.at[pl.ds(r0, CR)], bufx[b], sem_lx[b]).wait()
            pltpu.make_async_copy(pe_hbm.at[pl.ds(t0, CR)], bufp[b], sem_lp[b]).wait()

        def wait_store(i, b):
            r0 = base + i * CR
            pltpu.make_async_copy(bufx[b], out_hbm.at[pl.ds(r0, CR)], sem_st[b]).wait()

        for b in range(_LA):
            start_loads(b, b)

        def iteration(i, b):
            wait_loads(i, b)

            def row_body(r, carry2):
                for c in range(n_col):
                    v = bufp[b][r, pl.ds(c * L, L)]
                    plsc.addupdate(bufx[b].at[r, pl.ds(c * L, L)], v)
                return carry2

            lax.fori_loop(0, CR, row_body, 0)
            r0 = base + i * CR
            pltpu.async_copy(bufx[b], out_hbm.at[pl.ds(r0, CR)], sem_st[b])

            bn = (b + _LA) % _NB
            j = i + _LA

            def prefetch(_):
                lax.cond(i + _LA >= _NB, lambda __: wait_store(j - _NB, bn),
                         lambda __: None, 0)
                start_loads(j, bn)
                return 0

            lax.cond(j < n_chunks, prefetch, lambda _: 0, 0)

        def group(g, carry):
            for b in range(_NB):
                iteration(g * _NB + b, b)
            return carry

        lax.fori_loop(0, n_chunks // _NB, group, 0)

        for b in range(_NB):
            wait_store(n_chunks - _NB + b, (n_chunks - _NB + b) % _NB)

    return k


def kernel(x, pe_weight):
    B, T, D = x.shape
    R = B * T
    k = _make_sc_kernel(R, T, D)
    out = k(x.reshape(R, D), pe_weight)
    return out.reshape(B, T, D)


# speedup vs baseline: 2.5597x; 1.5849x over previous
import functools

import jax
import jax.numpy as jnp
from jax import lax
from jax.experimental import pallas as pl
from jax.experimental.pallas import tpu as pltpu
from jax.experimental.pallas import tpu_sc as plsc

_NB = 4
_LA = 2


def _make_sc_kernel(R, T, D):
    info = plsc.get_sparse_core_info()
    NC, NS, L = info.num_cores, info.num_subcores, info.num_lanes
    NW = NC * NS
    rows_per_w = R // NW
    CR = 16
    n_chunks = rows_per_w // CR
    n_col = D // L

    mesh = plsc.VectorSubcoreMesh(core_axis_name="c", subcore_axis_name="s")

    scratch = (
        [pltpu.VMEM((CR, D), jnp.float32) for _ in range(_NB)]
        + [pltpu.VMEM((CR, D), jnp.float32) for _ in range(_NB)]
        + [pltpu.SemaphoreType.DMA for _ in range(3 * _NB)]
    )

    @functools.partial(
        pl.kernel,
        mesh=mesh,
        out_type=jax.ShapeDtypeStruct((R, D), jnp.float32),
        scratch_types=scratch,
    )
    def k(x_hbm, pe_hbm, out_hbm, *refs):
        bufx = refs[:_NB]
        bufp = refs[_NB:2 * _NB]
        sem_lx = refs[2 * _NB:2 * _NB + _NB]
        sem_lp = refs[2 * _NB + _NB:2 * _NB + 2 * _NB]
        sem_st = refs[2 * _NB + 2 * _NB:]

        wid = lax.axis_index("s") * NC + lax.axis_index("c")
        base = wid * rows_per_w

        def start_loads(i, b):
            r0 = base + i * CR
            t0 = lax.rem(r0, T)
            pltpu.async_copy(x_hbm.at[pl.ds(r0, CR)], bufx[b], sem_lx[b])
            pltpu.async_copy(pe_hbm.at[pl.ds(t0, CR)], bufp[b], sem_lp[b])

        def wait_loads(i, b):
            r0 = base + i * CR
            t0 = lax.rem(r0, T)
            pltpu.make_async_copy(x_hbm.at[pl.ds(r0, CR)], bufx[b], sem_lx[b]).wait()
            pltpu.make_async_copy(pe_hbm.at[pl.ds(t0, CR)], bufp[b], sem_lp[b]).wait()

        def wait_store(i, b):
            r0 = base + i * CR
            pltpu.make_async_copy(bufx[b], out_hbm.at[pl.ds(r0, CR)], sem_st[b]).wait()

        for b in range(_LA):
            start_loads(b, b)

        def iteration(i, b):
            wait_loads(i, b)

            def row_body(r, carry2):
                G = 16
                for g in range(n_col // G):
                    vs = [bufp[b][r, pl.ds((g * G + u) * L, L)]
                          for u in range(G)]
                    for u in range(G):
                        plsc.addupdate(bufx[b].at[r, pl.ds((g * G + u) * L, L)],
                                       vs[u])
                return carry2

            lax.fori_loop(0, CR, row_body, 0)
            r0 = base + i * CR
            pltpu.async_copy(bufx[b], out_hbm.at[pl.ds(r0, CR)], sem_st[b])

            bn = (b + _LA) % _NB
            j = i + _LA

            def prefetch(_):
                lax.cond(i + _LA >= _NB, lambda __: wait_store(j - _NB, bn),
                         lambda __: None, 0)
                start_loads(j, bn)
                return 0

            lax.cond(j < n_chunks, prefetch, lambda _: 0, 0)

        def group(g, carry):
            for b in range(_NB):
                iteration(g * _NB + b, b)
            return carry

        lax.fori_loop(0, n_chunks // _NB, group, 0)

        for b in range(_NB):
            wait_store(n_chunks - _NB + b, (n_chunks - _NB + b) % _NB)

    return k


def kernel(x, pe_weight):
    B, T, D = x.shape
    R = B * T
    k = _make_sc_kernel(R, T, D)
    out = k(x.reshape(R, D), pe_weight)
    return out.reshape(B, T, D)
